# baseline (speedup 1.0000x reference)
import functools

import jax
import jax.numpy as jnp
from jax import lax
from jax.experimental import pallas as pl
from jax.experimental.pallas import tpu as pltpu
from jax.experimental.pallas import tpu_sc as plsc

N = 10000
NP = 10240
E = 320000
CH = 10000
NG = NP // 16
F32 = jnp.float32
I32 = jnp.int32

@functools.cache
def _mesh():
    return plsc.VectorSubcoreMesh(core_axis_name="c", subcore_axis_name="s")


def _zero_f32(ref, ngroups):
    def body(i, _):
        ref[pl.ds(i * 16, 16)] = jnp.zeros((16,), F32)
        return _
    lax.fori_loop(0, ngroups, body, None)


def _newton_rsqrt(x):
    i = plsc.bitcast(x, I32)
    i = jnp.int32(0x5F3759DF) - (i >> 1)
    y = plsc.bitcast(i, F32)
    for _ in range(3):
        y = y * (jnp.float32(1.5) - jnp.float32(0.5) * x * y * y)
    return y


def _ds8(off, n):
    return pl.ds(pl.multiple_of(off, 8), n)


def _fire(copies, sem):
    ds = [pltpu.async_copy(s, d, sem) for s, d in copies]
    for d in ds:
        d.wait()


def _edge_loop(r_buf, c_buf, f_buf, h_buf, acc, acc2, n_units):
    def body(i, _):
        base = i * 80
        for u in range(5):
            off = base + u * 16
            rv = r_buf[pl.ds(off, 16)]
            cv = c_buf[pl.ds(off, 16)]
            nv = f_buf[pl.ds(off, 16)]
            gv = plsc.load_gather(h_buf, [rv])
            plsc.addupdate_scatter(acc if u % 2 == 0 else acc2, [cv], gv * nv)
        return _
    lax.fori_loop(0, n_units, body, None)


def _merge_acc(acc, acc2, ngroups):
    def body(i, _):
        for u in range(5):
            off = i * 80 + u * 16
            acc[pl.ds(off, 16)] = acc[pl.ds(off, 16)] + acc2[pl.ds(off, 16)]
        return _
    lax.fori_loop(0, ngroups // 5, body, None)


def _tc_h1_body(x_ref, w_ref, o_ref):
    o_ref[...] = lax.dot_general(
        w_ref[...], x_ref[...], (((0,), (1,)), ((), ())),
        preferred_element_type=F32)


def _tc_h1(x_pad, W1):
    return pl.pallas_call(
        _tc_h1_body,
        out_shape=jax.ShapeDtypeStruct((W1.shape[1], NP), F32),
    )(x_pad, W1)


def _k1_body(row_h, col_h, ew_h, h1t_h,
             parts_h, norm_h, dinv_h,
             r_buf, c_buf, f_buf, dv_buf, h_buf, acc, acc2, s1, s2,
             parts_sh, dinv_sh, sem):
    c = lax.axis_index("c")
    s = lax.axis_index("s")

    _zero_f32(acc, NG)
    _zero_f32(acc2, NG)
    for k in range(2):
        base = s * 20000 + k * CH
        _fire([(col_h.at[_ds8(base, CH)], c_buf),
               (ew_h.at[_ds8(base, CH)], f_buf)], sem)

        def dbody(i, _):
            for u in range(5):
                off = i * 80 + u * 16
                cv = c_buf[pl.ds(off, 16)]
                wv = f_buf[pl.ds(off, 16)]
                plsc.addupdate_scatter(acc if u % 2 == 0 else acc2, [cv], wv)
            return _
        lax.fori_loop(0, CH // 80, dbody, None)
    _merge_acc(acc, acc2, NG)
    pltpu.sync_copy(acc, parts_sh.at[_ds8(s * NP, NP)])
    plsc.subcore_barrier()

    sl = s * 640
    pltpu.sync_copy(parts_sh.at[_ds8(sl, 640)], s2)
    for p in range(1, 16):
        pltpu.sync_copy(parts_sh.at[_ds8(p * NP + sl, 640)], s1)

        def abody(i, _):
            s2[pl.ds(i * 16, 16)] = s2[pl.ds(i * 16, 16)] + s1[pl.ds(i * 16, 16)]
            return _
        lax.fori_loop(0, 40, abody, None)

    def nbody(i, _):
        d = s2[pl.ds(i * 16, 16)] + jnp.float32(1.0)
        s2[pl.ds(i * 16, 16)] = _newton_rsqrt(d)
        return _
    lax.fori_loop(0, 40, nbody, None)
    pltpu.sync_copy(s2, dinv_sh.at[_ds8(sl, 640)])
    plsc.subcore_barrier()
    pltpu.sync_copy(dinv_sh, dv_buf)

    qbase = (c * 16 + s) * CH
    _fire([(row_h.at[_ds8(qbase, CH)], r_buf),
           (col_h.at[_ds8(qbase, CH)], c_buf),
           (ew_h.at[_ds8(qbase, CH)], f_buf)], sem)

    def cbody(i, _):
        for u in range(5):
            off = i * 80 + u * 16
            rv = r_buf[pl.ds(off, 16)]
            cv = c_buf[pl.ds(off, 16)]
            ev = f_buf[pl.ds(off, 16)]
            dr = plsc.load_gather(dv_buf, [rv])
            dc = plsc.load_gather(dv_buf, [cv])
            f_buf[pl.ds(off, 16)] = dr * ev * dc
        return _
    lax.fori_loop(0, CH // 80, cbody, None)
    pltpu.sync_copy(f_buf, norm_h.at[_ds8(qbase, CH)])

    @pl.when(jnp.logical_and(c == 0, s == 0))
    def _():
        pltpu.sync_copy(dv_buf, dinv_h)
    plsc.subcore_barrier()

    f = s % 8
    g = s // 8 + 2 * c
    pltpu.sync_copy(h1t_h.at[_ds8(f * NP, NP)], h_buf)
    _zero_f32(acc, NG)
    _zero_f32(acc2, NG)
    for k in range(8):
        base = (g * 8 + k) * CH
        _fire([(row_h.at[_ds8(base, CH)], r_buf),
               (col_h.at[_ds8(base, CH)], c_buf),
               (norm_h.at[_ds8(base, CH)], f_buf)], sem)
        _edge_loop(r_buf, c_buf, f_buf, h_buf, acc, acc2, CH // 80)
    _merge_acc(acc, acc2, NG)
    pltpu.sync_copy(acc, parts_h.at[_ds8((f * 4 + g) * NP, NP)])


def _k1(row, col, ew, h1t):
    fn = pl.kernel(
        _k1_body,
        out_type=[
            jax.ShapeDtypeStruct((8 * 4 * NP,), F32),
            jax.ShapeDtypeStruct((E,), F32),
            jax.ShapeDtypeStruct((NP,), F32),
        ],
        mesh=_mesh(),
        compiler_params=pltpu.CompilerParams(needs_layout_passes=False),
        scratch_types=[
            pltpu.VMEM((CH,), I32),
            pltpu.VMEM((CH,), I32),
            pltpu.VMEM((CH,), F32),
            pltpu.VMEM((NP,), F32),
            pltpu.VMEM((NP,), F32),
            pltpu.VMEM((NP,), F32),
            pltpu.VMEM((NP,), F32),
            pltpu.VMEM((640,), F32),
            pltpu.VMEM((640,), F32),
            pltpu.VMEM_SHARED((16 * NP,), F32),
            pltpu.VMEM_SHARED((NP,), F32),
            pltpu.SemaphoreType.DMA,
        ],
    )
    return fn(row, col, ew, h1t)


def _agg_body(F, G, row_h, col_h, norm_h, ht_h, parts_h,
              r_buf, c_buf, f_buf, h_buf, acc, sem):
    c = lax.axis_index("c")
    s = lax.axis_index("s")
    f = s % F
    g = s // F + (16 // F) * c
    eg = E // G
    pltpu.sync_copy(ht_h.at[_ds8(f * NP, NP)], h_buf)
    _zero_f32(acc, NG)
    for k in range(eg // CH):
        base = g * eg + k * CH
        _fire([(row_h.at[_ds8(base, CH)], r_buf),
               (col_h.at[_ds8(base, CH)], c_buf),
               (norm_h.at[_ds8(base, CH)], f_buf)], sem)
        _edge_loop(r_buf, c_buf, f_buf, h_buf, acc, CH // 80)
    pltpu.sync_copy(acc, parts_h.at[_ds8((f * G + g) * NP, NP)])


def _agg(F, G, row, col, norm, ht):
    fn = pl.kernel(
        functools.partial(_agg_body, F, G),
        out_type=[jax.ShapeDtypeStruct((F * G * NP,), F32)],
        mesh=_mesh(),
        compiler_params=pltpu.CompilerParams(needs_layout_passes=False),
        scratch_types=[
            pltpu.VMEM((CH,), I32),
            pltpu.VMEM((CH,), I32),
            pltpu.VMEM((CH,), F32),
            pltpu.VMEM((NP,), F32),
            pltpu.VMEM((NP,), F32),
            pltpu.SemaphoreType.DMA,
        ],
    )
    return fn(row, col, norm, ht)[0]


def _fused_body(Fp, Gp, F, G, row_h, col_h, norm_h, pprev_h, hprev_h, dinv_h,
                ws_h, bs_h, parts_h, hout_h,
                r_buf, c_buf, f_buf, h_buf, acc, acc2,
                pa_p, pa_h, pa_d, pa_o, w_buf, b_buf, h_sh, sem):
    c = lax.axis_index("c")
    s = lax.axis_index("s")

    nbase = s * 640
    copies = [(dinv_h.at[_ds8(nbase, 640)], pa_d), (bs_h, b_buf), (ws_h, w_buf)]
    for f in range(Fp):
        copies.append((hprev_h.at[_ds8(f * NP + nbase, 640)],
                       pa_h.at[pl.ds(f * 640, 640)]))
        for g in range(Gp):
            copies.append((pprev_h.at[_ds8((f * Gp + g) * NP + nbase, 640)],
                           pa_p.at[pl.ds((f * Gp + g) * 640, 640)]))
    _fire(copies, sem)

    def abody(i, _):
        off = i * 16
        dv = pa_d[pl.ds(off, 16)]
        dv2 = dv * dv
        outs = [jnp.zeros((16,), F32) for _ in range(F)]
        for f in range(Fp):
            comb = b_buf[pl.ds(f * 16, 16)]
            for g in range(Gp):
                comb = comb + pa_p[pl.ds((f * Gp + g) * 640 + off, 16)]
            comb = comb + dv2 * pa_h[pl.ds(f * 640 + off, 16)]
            r = jnp.maximum(comb, jnp.float32(0.0))
            for j in range(F):
                outs[j] = outs[j] + r * w_buf[pl.ds((f * F + j) * 16, 16)]
        for j in range(F):
            pa_o[pl.ds(j * 640 + off, 16)] = outs[j]
        return _
    lax.fori_loop(0, 40, abody, None)

    for j in range(F):
        pltpu.sync_copy(pa_o.at[pl.ds(j * 640, 640)],
                        h_sh.at[_ds8(j * NP + nbase, 640)])

    @pl.when(c == 0)
    def _():
        for j in range(F):
            pltpu.sync_copy(pa_o.at[pl.ds(j * 640, 640)],
                            hout_h.at[_ds8(j * NP + nbase, 640)])
    plsc.subcore_barrier()

    f = s % F
    g = s // F + (16 // F) * c
    eg = E // G
    pltpu.sync_copy(h_sh.at[_ds8(f * NP, NP)], h_buf)
    _zero_f32(acc, NG)
    _zero_f32(acc2, NG)
    for k in range(eg // CH):
        base = g * eg + k * CH
        _fire([(row_h.at[_ds8(base, CH)], r_buf),
               (col_h.at[_ds8(base, CH)], c_buf),
               (norm_h.at[_ds8(base, CH)], f_buf)], sem)
        _edge_loop(r_buf, c_buf, f_buf, h_buf, acc, acc2, CH // 80)
    _merge_acc(acc, acc2, NG)
    pltpu.sync_copy(acc, parts_h.at[_ds8((f * G + g) * NP, NP)])


def _fused(Fp, Gp, F, G, row, col, norm, pprev, hprev, dinv, ws, bs):
    fn = pl.kernel(
        functools.partial(_fused_body, Fp, Gp, F, G),
        out_type=[
            jax.ShapeDtypeStruct((F * G * NP,), F32),
            jax.ShapeDtypeStruct((F * NP,), F32),
        ],
        mesh=_mesh(),
        compiler_params=pltpu.CompilerParams(needs_layout_passes=False),
        scratch_types=[
            pltpu.VMEM((CH,), I32),
            pltpu.VMEM((CH,), I32),
            pltpu.VMEM((CH,), F32),
            pltpu.VMEM((NP,), F32),
            pltpu.VMEM((NP,), F32),
            pltpu.VMEM((NP,), F32),
            pltpu.VMEM((Fp * Gp * 640,), F32),
            pltpu.VMEM((Fp * 640,), F32),
            pltpu.VMEM((640,), F32),
            pltpu.VMEM((F * 640,), F32),
            pltpu.VMEM((Fp * F * 16,), F32),
            pltpu.VMEM((Fp * 16,), F32),
            pltpu.VMEM_SHARED((F * NP,), F32),
            pltpu.SemaphoreType.DMA,
        ],
    )
    return fn(row, col, norm, pprev, hprev, dinv, ws, bs)


def _dense_body(Fin, G, Fout, matmul,
                parts_h, ht_h, dinv_h, ws_h, bs_h, out_h,
                p_buf, h_buf, d_buf, w_buf, b_buf, o_buf, sem):
    c = lax.axis_index("c")
    s = lax.axis_index("s")
    wid = c * 16 + s
    nbase = wid * (NP // 32)
    copies = [(dinv_h.at[_ds8(nbase, 320)], d_buf), (bs_h, b_buf)]
    if matmul:
        copies.append((ws_h, w_buf))
    for f in range(Fin):
        copies.append((ht_h.at[_ds8(f * NP + nbase, 320)],
                       h_buf.at[pl.ds(f * 320, 320)]))
        for g in range(G):
            copies.append((parts_h.at[_ds8((f * G + g) * NP + nbase, 320)],
                           p_buf.at[pl.ds((f * G + g) * 320, 320)]))
    _fire(copies, sem)

    def body(i, _):
        off = i * 16
        dv = d_buf[pl.ds(off, 16)]
        dv2 = dv * dv
        outs = [jnp.zeros((16,), F32) for _ in range(Fout)]
        for f in range(Fin):
            comb = b_buf[pl.ds(f * 16, 16)]
            for g in range(G):
                comb = comb + p_buf[pl.ds((f * G + g) * 320 + off, 16)]
            comb = comb + dv2 * h_buf[pl.ds(f * 320 + off, 16)]
            if matmul:
                r = jnp.maximum(comb, jnp.float32(0.0))
                for j in range(Fout):
                    outs[j] = outs[j] + r * w_buf[pl.ds((f * Fout + j) * 16, 16)]
            else:
                outs[f] = comb
        for j in range(Fout):
            o_buf[pl.ds(j * 320 + off, 16)] = outs[j]
        return _
    lax.fori_loop(0, 20, body, None)
    for j in range(Fout):
        pltpu.sync_copy(o_buf.at[pl.ds(j * 320, 320)],
                        out_h.at[_ds8(j * NP + nbase, 320)])


def _dense(Fin, G, Fout, matmul, parts, ht, dinv, ws, bs):
    fn = pl.kernel(
        functools.partial(_dense_body, Fin, G, Fout, matmul),
        out_type=[jax.ShapeDtypeStruct((Fout * NP,), F32)],
        mesh=_mesh(),
        compiler_params=pltpu.CompilerParams(needs_layout_passes=False),
        scratch_types=[
            pltpu.VMEM((Fin * G * 320,), F32),
            pltpu.VMEM((Fin * 320,), F32),
            pltpu.VMEM((320,), F32),
            pltpu.VMEM((max(Fin * Fout * 16, 16),), F32),
            pltpu.VMEM((Fin * 16,), F32),
            pltpu.VMEM((Fout * 320,), F32),
            pltpu.SemaphoreType.DMA,
        ],
    )
    return fn(parts, ht, dinv, ws, bs)[0]


def kernel(x, edge_index, edge_attr, W1, b1, W2, b2, W3, b3):
    row = edge_index[0].astype(I32)
    col = edge_index[1].astype(I32)
    ew = edge_attr

    x_pad = jnp.pad(x, ((0, NP - N), (0, 0)))
    h1t = _tc_h1(x_pad, W1).reshape(-1)

    w2s = jnp.repeat(W2.reshape(-1), 16)
    w3s = jnp.repeat(W3.reshape(-1), 16)
    b1s = jnp.repeat(b1, 16)
    b2s = jnp.repeat(b2, 16)
    b3s = jnp.repeat(b3, 16)

    parts1, norm, dinv = _k1(row, col, ew, h1t)
    parts2, h2t = _fused(8, 4, 4, 8, row, col, norm, parts1, h1t, dinv,
                         w2s, b1s)
    parts3, h3t = _fused(4, 8, 2, 16, row, col, norm, parts2, h2t, dinv,
                         w3s, b2s)
    outt = _dense(2, 16, 2, False, parts3, h3t, dinv, w3s, b3s)
    return outt.reshape(2, NP)[:, :N].T


# speedup vs baseline: 1.1981x; 1.1981x over previous
import functools

import jax
import jax.numpy as jnp
from jax import lax
from jax.experimental import pallas as pl
from jax.experimental.pallas import tpu as pltpu
from jax.experimental.pallas import tpu_sc as plsc

N = 10000
NP = 10240
E = 320000
CH = 10000
NG = NP // 16
F32 = jnp.float32
I32 = jnp.int32


@functools.cache
def _mesh():
    return plsc.VectorSubcoreMesh(core_axis_name="c", subcore_axis_name="s")


def _zero_f32(ref, ngroups):
    def body(i, _):
        ref[pl.ds(i * 16, 16)] = jnp.zeros((16,), F32)
        return _
    lax.fori_loop(0, ngroups, body, None)


def _newton_rsqrt(x):
    i = plsc.bitcast(x, I32)
    i = jnp.int32(0x5F3759DF) - (i >> 1)
    y = plsc.bitcast(i, F32)
    for _ in range(3):
        y = y * (jnp.float32(1.5) - jnp.float32(0.5) * x * y * y)
    return y


def _ds8(off, n):
    return pl.ds(pl.multiple_of(off, 8), n)


def _fire(copies, sem):
    ds = [pltpu.async_copy(s, d, sem) for s, d in copies]
    for d in ds:
        d.wait()


def _scale_by(ref, scale_ref, ngroups):
    def body(i, _):
        sl = pl.ds(i * 16, 16)
        ref[sl] = ref[sl] * scale_ref[sl]
        return _
    lax.fori_loop(0, ngroups, body, None)


def _edge_loop(r_buf, c_buf, f_buf, h_buf, acc, n_units):
    def body(i, _):
        base = i * 80
        for u in range(5):
            off = base + u * 16
            rv = r_buf[pl.ds(off, 16)]
            cv = c_buf[pl.ds(off, 16)]
            nv = f_buf[pl.ds(off, 16)]
            gv = plsc.load_gather(h_buf, [rv])
            plsc.addupdate_scatter(acc, [cv], gv * nv)
        return _
    lax.fori_loop(0, n_units, body, None)


def _chunked_edges(row_h, col_h, ew_h, bufs, sem, bases, inner):
    def fire(k, slot):
        r_buf, c_buf, f_buf = bufs[slot]
        return [pltpu.async_copy(row_h.at[_ds8(bases[k], CH)], r_buf, sem),
                pltpu.async_copy(col_h.at[_ds8(bases[k], CH)], c_buf, sem),
                pltpu.async_copy(ew_h.at[_ds8(bases[k], CH)], f_buf, sem)]
    pending = fire(0, 0)
    for k in range(len(bases)):
        for d in pending:
            d.wait()
        if k + 1 < len(bases):
            nxt = fire(k + 1, (k + 1) % 2)
        inner(*bufs[k % 2])
        if k + 1 < len(bases):
            pending = nxt


def _tc_h1_body(x_ref, w_ref, o_ref):
    o_ref[...] = lax.dot_general(
        w_ref[...], x_ref[...], (((0,), (1,)), ((), ())),
        preferred_element_type=F32)


def _tc_h1(x_pad, W1):
    return pl.pallas_call(
        _tc_h1_body,
        out_shape=jax.ShapeDtypeStruct((W1.shape[1], NP), F32),
    )(x_pad, W1)


def _k1_body(row_h, col_h, ew_h, h1t_h,
             parts_h, dinv_h, h1s_h,
             r_buf, c_buf, f_buf, r2_buf, c2_buf, f2_buf,
             dv_buf, h_buf, acc, s1, s2,
             parts_sh, dinv_sh, sem):
    c = lax.axis_index("c")
    s = lax.axis_index("s")
    bufs = ((r_buf, c_buf, f_buf), (r2_buf, c2_buf, f2_buf))

    _zero_f32(dv_buf, NG)

    def deg_inner(_r, cb, fb):
        def dbody(i, _):
            for u in range(5):
                off = i * 80 + u * 16
                cv = cb[pl.ds(off, 16)]
                wv = fb[pl.ds(off, 16)]
                plsc.addupdate_scatter(dv_buf, [cv], wv)
            return _
        lax.fori_loop(0, CH // 80, dbody, None)
    _chunked_edges(row_h, col_h, ew_h, bufs, sem,
                   [s * 20000, s * 20000 + CH], deg_inner)
    pltpu.sync_copy(dv_buf, parts_sh.at[_ds8(s * NP, NP)])
    plsc.subcore_barrier()

    sl = s * 640
    pltpu.sync_copy(parts_sh.at[_ds8(sl, 640)], s2)
    for p in range(1, 16):
        pltpu.sync_copy(parts_sh.at[_ds8(p * NP + sl, 640)], s1)

        def abody(i, _):
            s2[pl.ds(i * 16, 16)] = s2[pl.ds(i * 16, 16)] + s1[pl.ds(i * 16, 16)]
            return _
        lax.fori_loop(0, 40, abody, None)

    def nbody(i, _):
        d = s2[pl.ds(i * 16, 16)] + jnp.float32(1.0)
        s2[pl.ds(i * 16, 16)] = _newton_rsqrt(d)
        return _
    lax.fori_loop(0, 40, nbody, None)
    pltpu.sync_copy(s2, dinv_sh.at[_ds8(sl, 640)])
    plsc.subcore_barrier()
    pltpu.sync_copy(dinv_sh, dv_buf)

    @pl.when(jnp.logical_and(c == 0, s == 0))
    def _():
        pltpu.sync_copy(dv_buf, dinv_h)

    f = s % 8
    g = s // 8 + 2 * c
    pltpu.sync_copy(h1t_h.at[_ds8(f * NP, NP)], h_buf)
    _scale_by(h_buf, dv_buf, NG)

    @pl.when(jnp.logical_and(c == 0, s < 8))
    def _():
        pltpu.sync_copy(h_buf, h1s_h.at[_ds8(f * NP, NP)])
    _zero_f32(acc, NG)

    def agg_inner(rb, cb, fb):
        _edge_loop(rb, cb, fb, h_buf, acc, CH // 80)
    _chunked_edges(row_h, col_h, ew_h, bufs, sem,
                   [(g * 8 + k) * CH for k in range(8)], agg_inner)
    pltpu.sync_copy(acc, parts_h.at[_ds8((f * 4 + g) * NP, NP)])


def _k1(row, col, ew, h1t):
    fn = pl.kernel(
        _k1_body,
        out_type=[
            jax.ShapeDtypeStruct((8 * 4 * NP,), F32),
            jax.ShapeDtypeStruct((NP,), F32),
            jax.ShapeDtypeStruct((8 * NP,), F32),
        ],
        mesh=_mesh(),
        compiler_params=pltpu.CompilerParams(needs_layout_passes=False),
        scratch_types=[
            pltpu.VMEM((CH,), I32),
            pltpu.VMEM((CH,), I32),
            pltpu.VMEM((CH,), F32),
            pltpu.VMEM((CH,), I32),
            pltpu.VMEM((CH,), I32),
            pltpu.VMEM((CH,), F32),
            pltpu.VMEM((NP,), F32),
            pltpu.VMEM((NP,), F32),
            pltpu.VMEM((NP,), F32),
            pltpu.VMEM((640,), F32),
            pltpu.VMEM((640,), F32),
            pltpu.VMEM_SHARED((16 * NP,), F32),
            pltpu.VMEM_SHARED((NP,), F32),
            pltpu.SemaphoreType.DMA,
        ],
    )
    return fn(row, col, ew, h1t)


def _fused_body(Fp, Gp, F, G, row_h, col_h, ew_h, pprev_h, hprev_h, dinv_h,
                ws_h, bs_h, parts_h, hout_h,
                r_buf, c_buf, f_buf, r2_buf, c2_buf, f2_buf, h_buf, acc,
                pa_p, pa_h, pa_d, pa_o, w_buf, b_buf, h_sh, sem):
    c = lax.axis_index("c")
    s = lax.axis_index("s")
    bufs = ((r_buf, c_buf, f_buf), (r2_buf, c2_buf, f2_buf))

    nbase = s * 640
    copies = [(dinv_h.at[_ds8(nbase, 640)], pa_d), (bs_h, b_buf), (ws_h, w_buf)]
    for f in range(Fp):
        copies.append((hprev_h.at[_ds8(f * NP + nbase, 640)],
                       pa_h.at[pl.ds(f * 640, 640)]))
        for g in range(Gp):
            copies.append((pprev_h.at[_ds8((f * Gp + g) * NP + nbase, 640)],
                           pa_p.at[pl.ds((f * Gp + g) * 640, 640)]))
    _fire(copies, sem)

    def abody(i, _):
        off = i * 16
        dv = pa_d[pl.ds(off, 16)]
        outs = [jnp.zeros((16,), F32) for _ in range(F)]
        for f in range(Fp):
            t = pa_h[pl.ds(f * 640 + off, 16)]
            for g in range(Gp):
                t = t + pa_p[pl.ds((f * Gp + g) * 640 + off, 16)]
            o = dv * t + b_buf[pl.ds(f * 16, 16)]
            r = jnp.maximum(o, jnp.float32(0.0))
            for j in range(F):
                outs[j] = outs[j] + r * w_buf[pl.ds((f * F + j) * 16, 16)]
        for j in range(F):
            pa_o[pl.ds(j * 640 + off, 16)] = dv * outs[j]
        return _
    lax.fori_loop(0, 40, abody, None)

    for j in range(F):
        pltpu.sync_copy(pa_o.at[pl.ds(j * 640, 640)],
                        h_sh.at[_ds8(j * NP + nbase, 640)])

    @pl.when(c == 0)
    def _():
        for j in range(F):
            pltpu.sync_copy(pa_o.at[pl.ds(j * 640, 640)],
                            hout_h.at[_ds8(j * NP + nbase, 640)])
    plsc.subcore_barrier()

    f = s % F
    g = s // F + (16 // F) * c
    eg = E // G
    pltpu.sync_copy(h_sh.at[_ds8(f * NP, NP)], h_buf)
    _zero_f32(acc, NG)

    def agg_inner(rb, cb, fb):
        _edge_loop(rb, cb, fb, h_buf, acc, CH // 80)
    _chunked_edges(row_h, col_h, ew_h, bufs, sem,
                   [g * eg + k * CH for k in range(eg // CH)], agg_inner)
    pltpu.sync_copy(acc, parts_h.at[_ds8((f * G + g) * NP, NP)])


def _fused(Fp, Gp, F, G, row, col, ew, pprev, hprev, dinv, ws, bs):
    fn = pl.kernel(
        functools.partial(_fused_body, Fp, Gp, F, G),
        out_type=[
            jax.ShapeDtypeStruct((F * G * NP,), F32),
            jax.ShapeDtypeStruct((F * NP,), F32),
        ],
        mesh=_mesh(),
        compiler_params=pltpu.CompilerParams(needs_layout_passes=False),
        scratch_types=[
            pltpu.VMEM((CH,), I32),
            pltpu.VMEM((CH,), I32),
            pltpu.VMEM((CH,), F32),
            pltpu.VMEM((CH,), I32),
            pltpu.VMEM((CH,), I32),
            pltpu.VMEM((CH,), F32),
            pltpu.VMEM((NP,), F32),
            pltpu.VMEM((NP,), F32),
            pltpu.VMEM((Fp * Gp * 640,), F32),
            pltpu.VMEM((Fp * 640,), F32),
            pltpu.VMEM((640,), F32),
            pltpu.VMEM((F * 640,), F32),
            pltpu.VMEM((Fp * F * 16,), F32),
            pltpu.VMEM((Fp * 16,), F32),
            pltpu.VMEM_SHARED((F * NP,), F32),
            pltpu.SemaphoreType.DMA,
        ],
    )
    return fn(row, col, ew, pprev, hprev, dinv, ws, bs)


def _final_body(Fin, G, parts_h, ht_h, dinv_h, bs_h, out_h,
                p_buf, h_buf, d_buf, b_buf, o_buf, sem):
    c = lax.axis_index("c")
    s = lax.axis_index("s")
    wid = c * 16 + s
    nbase = wid * (NP // 32)
    copies = [(dinv_h.at[_ds8(nbase, 320)], d_buf), (bs_h, b_buf)]
    for f in range(Fin):
        copies.append((ht_h.at[_ds8(f * NP + nbase, 320)],
                       h_buf.at[pl.ds(f * 320, 320)]))
        for g in range(G):
            copies.append((parts_h.at[_ds8((f * G + g) * NP + nbase, 320)],
                           p_buf.at[pl.ds((f * G + g) * 320, 320)]))
    _fire(copies, sem)

    def body(i, _):
        off = i * 16
        dv = d_buf[pl.ds(off, 16)]
        for f in range(Fin):
            t = h_buf[pl.ds(f * 320 + off, 16)]
            for g in range(G):
                t = t + p_buf[pl.ds((f * G + g) * 320 + off, 16)]
            o_buf[pl.ds(f * 320 + off, 16)] = dv * t + b_buf[pl.ds(f * 16, 16)]
        return _
    lax.fori_loop(0, 20, body, None)
    for f in range(Fin):
        pltpu.sync_copy(o_buf.at[pl.ds(f * 320, 320)],
                        out_h.at[_ds8(f * NP + nbase, 320)])


def _final(Fin, G, parts, ht, dinv, bs):
    fn = pl.kernel(
        functools.partial(_final_body, Fin, G),
        out_type=[jax.ShapeDtypeStruct((Fin * NP,), F32)],
        mesh=_mesh(),
        compiler_params=pltpu.CompilerParams(needs_layout_passes=False),
        scratch_types=[
            pltpu.VMEM((Fin * G * 320,), F32),
            pltpu.VMEM((Fin * 320,), F32),
            pltpu.VMEM((320,), F32),
            pltpu.VMEM((Fin * 16,), F32),
            pltpu.VMEM((Fin * 320,), F32),
            pltpu.SemaphoreType.DMA,
        ],
    )
    return fn(parts, ht, dinv, bs)[0]


def kernel(x, edge_index, edge_attr, W1, b1, W2, b2, W3, b3):
    row = edge_index[0].astype(I32)
    col = edge_index[1].astype(I32)
    ew = edge_attr

    x_pad = jnp.pad(x, ((0, NP - N), (0, 0)))
    h1t = _tc_h1(x_pad, W1).reshape(-1)

    w2s = jnp.repeat(W2.reshape(-1), 16)
    w3s = jnp.repeat(W3.reshape(-1), 16)
    b1s = jnp.repeat(b1, 16)
    b2s = jnp.repeat(b2, 16)
    b3s = jnp.repeat(b3, 16)

    parts1, dinv, h1s = _k1(row, col, ew, h1t)
    parts2, h2t = _fused(8, 4, 4, 8, row, col, ew, parts1, h1s, dinv,
                         w2s, b1s)
    parts3, h3t = _fused(4, 8, 2, 16, row, col, ew, parts2, h2t, dinv,
                         w3s, b2s)
    outt = _final(2, 16, parts3, h3t, dinv, b3s)
    return outt.reshape(2, NP)[:, :N].T
